# baseline (speedup 1.0000x reference)
import functools

import jax
import jax.numpy as jnp
from jax import lax
from jax.experimental import pallas as pl
from jax.experimental.pallas import tpu as pltpu
from jax.experimental.pallas import tpu_sc as plsc

B = 8192
N = 64
F = 16
H = 64
C = 128

BB = 256


W2N = 2 * N


def _gather_ego_rows(adj_wide, gidx2):
    info = plsc.get_sparse_core_info()
    nw = info.num_cores * info.num_subcores
    bpw = B // nw
    mesh = plsc.VectorSubcoreMesh(core_axis_name="c", subcore_axis_name="s")

    @functools.partial(
        pl.kernel,
        mesh=mesh,
        out_type=jax.ShapeDtypeStruct((B, W2N), jnp.float32),
        scratch_types=[
            pltpu.VMEM((bpw,), jnp.int32),
            pltpu.VMEM((bpw, W2N), jnp.float32),
            pltpu.SemaphoreType.DMA,
        ],
    )
    def gather_kernel(table_hbm, idx_hbm, out_hbm, idx_v, rows_v, sem):
        wid = lax.axis_index("s") * info.num_cores + lax.axis_index("c")
        base = wid * bpw
        pltpu.sync_copy(idx_hbm.at[pl.ds(base, bpw)], idx_v)
        pltpu.async_copy(table_hbm.at[idx_v], rows_v, sem).wait()
        pltpu.sync_copy(rows_v, out_hbm.at[pl.ds(base, bpw)])

    return gather_kernel(adj_wide, gidx2)


def _tc_body(cent_ref, nobs_ref, arow_ref, aid_ref,
             we_ref, be_ref, wm_ref, ws_ref,
             w1c_ref, w1h_ref, b1_ref, w2_ref, b2_ref, wvt_ref, bv_ref,
             out_ref):
    f32 = jnp.float32
    nobs = nobs_ref[...].reshape(BB * N, F)
    h = jax.nn.relu(
        jnp.dot(nobs, we_ref[...], preferred_element_type=f32) + be_ref[...]
    )
    h3 = h.reshape(BB, N, H)
    aid = aid_ref[...]
    wide = arow_ref[...]
    arow = jnp.where((aid % 2) == 0, wide[:, :N], wide[:, N:])
    deg = jnp.sum(arow, axis=1, keepdims=True)
    a = arow / (deg + 1e-6)
    onehot = (lax.broadcasted_iota(jnp.int32, (BB, N), 1) == aid).astype(f32)
    m = jnp.sum(a[:, :, None] * h3, axis=1)
    ego = jnp.sum(onehot[:, :, None] * h3, axis=1)
    nbd = jax.nn.relu(
        jnp.dot(m, wm_ref[...], preferred_element_type=f32)
        + jnp.dot(ego, ws_ref[...], preferred_element_type=f32)
    )
    x = jax.nn.relu(
        jnp.dot(cent_ref[...], w1c_ref[...], preferred_element_type=f32)
        + jnp.dot(nbd, w1h_ref[...], preferred_element_type=f32)
        + b1_ref[...]
    )
    x = jax.nn.relu(jnp.dot(x, w2_ref[...], preferred_element_type=f32) + b2_ref[...])
    out_ref[...] = jnp.sum(x * wvt_ref[...], axis=1, keepdims=True) + bv_ref[...]


def _critic_pallas(cent_obs, node_obs, adj_rows, aid,
                   W_embed, b_embed, W_msg, W_self,
                   W1c, W1h, b1, W2, b2, Wv_t, bv, interpret=False):
    grid = (B // BB,)
    full = lambda *s: pl.BlockSpec(s, lambda i: (0,) * len(s))
    return pl.pallas_call(
        _tc_body,
        grid=grid,
        in_specs=[
            pl.BlockSpec((BB, C), lambda i: (i, 0)),
            pl.BlockSpec((BB, N, F), lambda i: (i, 0, 0)),
            pl.BlockSpec((BB, W2N), lambda i: (i, 0)),
            pl.BlockSpec((BB, 1), lambda i: (i, 0)),
            full(F, H),
            full(1, H),
            full(H, H),
            full(H, H),
            full(C, H),
            full(H, H),
            full(1, H),
            full(H, H),
            full(1, H),
            full(1, H),
            full(1, 1),
        ],
        out_specs=pl.BlockSpec((BB, 1), lambda i: (i, 0)),
        out_shape=jax.ShapeDtypeStruct((B, 1), jnp.float32),
        interpret=interpret,
    )(cent_obs, node_obs, adj_rows, aid,
      W_embed, b_embed, W_msg, W_self,
      W1c, W1h, b1, W2, b2, Wv_t, bv)


def kernel(cent_obs, node_obs, adj, agent_id, rnn_states, masks,
           W_embed, b_embed, W_msg, W_self, W1, b1, W2, b2, Wv, bv):
    aid = agent_id[:, :1].astype(jnp.int32)
    gidx2 = (jnp.arange(B, dtype=jnp.int32) * N + aid[:, 0]) // 2
    adj_rows = _gather_ego_rows(adj.reshape(B * N * N // W2N, W2N), gidx2)
    values = _critic_pallas(
        cent_obs, node_obs, adj_rows, aid,
        W_embed, b_embed.reshape(1, H), W_msg, W_self,
        W1[:C], W1[C:], b1.reshape(1, H), W2, b2.reshape(1, H),
        Wv.reshape(1, H), bv.reshape(1, 1),
    )
    return values, rnn_states


# speedup vs baseline: 1.4655x; 1.4655x over previous
import functools

import jax
import jax.numpy as jnp
from jax import lax
from jax.experimental import pallas as pl
from jax.experimental.pallas import tpu as pltpu
from jax.experimental.pallas import tpu_sc as plsc

B = 8192
N = 64
F = 16
H = 64
C = 128

BB = 256


SUB = 8
LANES = 16
UN = 16


def _gather_ego_rows(adj_tiles, gidx):
    info = plsc.get_sparse_core_info()
    nw = info.num_cores * info.num_subcores
    bpw = B // nw
    mesh = plsc.VectorSubcoreMesh(core_axis_name="c", subcore_axis_name="s")

    @functools.partial(
        pl.kernel,
        mesh=mesh,
        out_type=jax.ShapeDtypeStruct((B, N), jnp.float32),
        scratch_types=[
            pltpu.VMEM((bpw,), jnp.int32),
            pltpu.VMEM((UN, SUB, N), jnp.float32),
            pltpu.VMEM((UN, N), jnp.float32),
            pltpu.SemaphoreType.DMA,
        ],
        compiler_params=pltpu.CompilerParams(use_tc_tiling_on_sc=True),
    )
    def gather_kernel(table_hbm, idx_hbm, out_hbm, idx_v, tiles_v, rows_v, sem):
        wid = lax.axis_index("s") * info.num_cores + lax.axis_index("c")
        base = wid * bpw
        pltpu.sync_copy(idx_hbm.at[pl.ds(base, bpw)], idx_v)

        def chunk(c, carry):
            off = c * UN
            vec = idx_v[pl.ds(off, UN)]
            for j in range(UN):
                pltpu.async_copy(table_hbm.at[vec[j] // SUB], tiles_v.at[j], sem)
            for j in range(UN):
                pltpu.make_async_copy(table_hbm.at[0], tiles_v.at[j], sem).wait()
            for j in range(UN):
                r = vec[j] % SUB
                for l in range(N // LANES):
                    rows_v[j, pl.ds(l * LANES, LANES)] = tiles_v[j, r, pl.ds(l * LANES, LANES)]
            pltpu.sync_copy(rows_v, out_hbm.at[pl.ds(base + off, UN)])
            return carry

        lax.fori_loop(0, bpw // UN, chunk, 0)

    return gather_kernel(adj_tiles, gidx)


def _tc_body(cent_ref, nobs_ref, arow_ref, aid_ref,
             we_ref, be_ref, wm_ref, ws_ref,
             w1c_ref, w1h_ref, b1_ref, w2_ref, b2_ref, wvt_ref, bv_ref,
             out_ref):
    f32 = jnp.float32
    nobs = nobs_ref[...].reshape(BB * N, F)
    h = jax.nn.relu(
        jnp.dot(nobs, we_ref[...], preferred_element_type=f32) + be_ref[...]
    )
    h3 = h.reshape(BB, N, H)
    aid = aid_ref[...]
    arow = arow_ref[...]
    deg = jnp.sum(arow, axis=1, keepdims=True)
    a = arow / (deg + 1e-6)
    onehot = (lax.broadcasted_iota(jnp.int32, (BB, N), 1) == aid).astype(f32)
    m = jnp.sum(a[:, :, None] * h3, axis=1)
    ego = jnp.sum(onehot[:, :, None] * h3, axis=1)
    nbd = jax.nn.relu(
        jnp.dot(m, wm_ref[...], preferred_element_type=f32)
        + jnp.dot(ego, ws_ref[...], preferred_element_type=f32)
    )
    x = jax.nn.relu(
        jnp.dot(cent_ref[...], w1c_ref[...], preferred_element_type=f32)
        + jnp.dot(nbd, w1h_ref[...], preferred_element_type=f32)
        + b1_ref[...]
    )
    x = jax.nn.relu(jnp.dot(x, w2_ref[...], preferred_element_type=f32) + b2_ref[...])
    out_ref[...] = jnp.sum(x * wvt_ref[...], axis=1, keepdims=True) + bv_ref[...]


def _critic_pallas(cent_obs, node_obs, adj_rows, aid,
                   W_embed, b_embed, W_msg, W_self,
                   W1c, W1h, b1, W2, b2, Wv_t, bv, interpret=False):
    grid = (B // BB,)
    full = lambda *s: pl.BlockSpec(s, lambda i: (0,) * len(s))
    return pl.pallas_call(
        _tc_body,
        grid=grid,
        in_specs=[
            pl.BlockSpec((BB, C), lambda i: (i, 0)),
            pl.BlockSpec((BB, N, F), lambda i: (i, 0, 0)),
            pl.BlockSpec((BB, N), lambda i: (i, 0)),
            pl.BlockSpec((BB, 1), lambda i: (i, 0)),
            full(F, H),
            full(1, H),
            full(H, H),
            full(H, H),
            full(C, H),
            full(H, H),
            full(1, H),
            full(H, H),
            full(1, H),
            full(1, H),
            full(1, 1),
        ],
        out_specs=pl.BlockSpec((BB, 1), lambda i: (i, 0)),
        out_shape=jax.ShapeDtypeStruct((B, 1), jnp.float32),
        interpret=interpret,
    )(cent_obs, node_obs, adj_rows, aid,
      W_embed, b_embed, W_msg, W_self,
      W1c, W1h, b1, W2, b2, Wv_t, bv)


def kernel(cent_obs, node_obs, adj, agent_id, rnn_states, masks,
           W_embed, b_embed, W_msg, W_self, W1, b1, W2, b2, Wv, bv):
    aid = agent_id[:, :1].astype(jnp.int32)
    gidx = jnp.arange(B, dtype=jnp.int32) * N + aid[:, 0]
    adj_rows = _gather_ego_rows(adj.reshape(B * N // SUB, SUB, N), gidx)
    values = _critic_pallas(
        cent_obs, node_obs, adj_rows, aid,
        W_embed, b_embed.reshape(1, H), W_msg, W_self,
        W1[:C], W1[C:], b1.reshape(1, H), W2, b2.reshape(1, H),
        Wv.reshape(1, H), bv.reshape(1, 1),
    )
    return values, rnn_states


# speedup vs baseline: 7.7247x; 5.2711x over previous
import jax
import jax.numpy as jnp
from jax import lax
from jax.experimental import pallas as pl

B = 8192
N = 64
F = 16
H = 64
C = 128

BL = 512


def _f32dot(a, b, dims):
    return lax.dot_general(a, b, (dims, ((), ())),
                           preferred_element_type=jnp.float32)


def _tc_body(adjT_ref, nobsT_ref, aidT_ref, cent_ref,
             we_ref, be_ref, wms_ref,
             w1c_ref, w1h_ref, b1_ref, w2_ref, b2_ref, wv_ref, bv_ref,
             out_ref):
    f32 = jnp.float32
    aid = aidT_ref[...]
    mask = (lax.broadcasted_iota(jnp.int32, (N, BL), 0) == aid).astype(f32)

    adjT = adjT_ref[...]
    arow = jnp.zeros((N, BL), f32)
    for n1 in range(N):
        arow = arow + mask[n1:n1 + 1, :] * adjT[n1]
    deg = jnp.sum(arow, axis=0, keepdims=True)
    aT = arow / (deg + 1e-6)

    nobsT = nobsT_ref[...]
    we = we_ref[...]
    be = be_ref[...]
    m = jnp.zeros((H, BL), f32)
    ego = jnp.zeros((H, BL), f32)
    for n in range(N):
        h_n = jax.nn.relu(_f32dot(we, nobsT[n], ((0,), (0,))) + be)
        m = m + aT[n:n + 1, :] * h_n
        ego = ego + mask[n:n + 1, :] * h_n

    p = jnp.concatenate([m, ego], axis=0)
    nbd = jax.nn.relu(_f32dot(p, wms_ref[...], ((0,), (0,))))

    x = jax.nn.relu(
        _f32dot(cent_ref[...], w1c_ref[...], ((1,), (0,)))
        + _f32dot(nbd, w1h_ref[...], ((1,), (0,)))
        + b1_ref[...]
    )
    x = jax.nn.relu(_f32dot(x, w2_ref[...], ((1,), (0,))) + b2_ref[...])
    out_ref[...] = jnp.sum(x * wv_ref[...], axis=1, keepdims=True) + bv_ref[...]


def kernel(cent_obs, node_obs, adj, agent_id, rnn_states, masks,
           W_embed, b_embed, W_msg, W_self, W1, b1, W2, b2, Wv, bv):
    adjT = jnp.transpose(adj, (1, 2, 0))
    nobsT = jnp.transpose(node_obs, (1, 2, 0))
    aidT = agent_id.astype(jnp.int32).reshape(1, B)
    wms = jnp.concatenate([W_msg, W_self], axis=0)

    grid = (B // BL,)
    full = lambda *s: pl.BlockSpec(s, lambda i: (0,) * len(s))
    values = pl.pallas_call(
        _tc_body,
        grid=grid,
        in_specs=[
            pl.BlockSpec((N, N, BL), lambda i: (0, 0, i)),
            pl.BlockSpec((N, F, BL), lambda i: (0, 0, i)),
            pl.BlockSpec((1, BL), lambda i: (0, i)),
            pl.BlockSpec((BL, C), lambda i: (i, 0)),
            full(F, H),
            full(H, 1),
            full(2 * H, H),
            full(C, H),
            full(H, H),
            full(1, H),
            full(H, H),
            full(1, H),
            full(1, H),
            full(1, 1),
        ],
        out_specs=pl.BlockSpec((BL, 1), lambda i: (i, 0)),
        out_shape=jax.ShapeDtypeStruct((B, 1), jnp.float32),
    )(adjT, nobsT, aidT, cent_obs,
      W_embed, b_embed.reshape(H, 1), wms,
      W1[:C], W1[C:], b1.reshape(1, H), W2, b2.reshape(1, H),
      Wv.reshape(1, H), bv.reshape(1, 1))
    return values, rnn_states


# speedup vs baseline: 8.5863x; 1.1115x over previous
import jax
import jax.numpy as jnp
from jax import lax
from jax.experimental import pallas as pl

B = 8192
N = 64
F = 16
H = 64
C = 128

BL = 512


def _f32dot(a, b, dims):
    return lax.dot_general(a, b, (dims, ((), ())),
                           preferred_element_type=jnp.float32)


def _tc_body(adjT_ref, nobsT_ref, aidT_ref, cent_ref,
             we_ref, be_ref, wms_ref,
             w1c_ref, w1h_ref, b1_ref, w2_ref, b2_ref, wv_ref, bv_ref,
             out_ref):
    f32 = jnp.float32
    aid = aidT_ref[...]
    mask = (lax.broadcasted_iota(jnp.int32, (N, BL), 0) == aid).astype(f32)

    adjT = adjT_ref[...]
    arow = jnp.zeros((N, BL), f32)
    for n1 in range(N):
        arow = arow + mask[n1:n1 + 1, :] * adjT[n1]
    deg = jnp.sum(arow, axis=0, keepdims=True)
    aT = arow / (deg + 1e-6)

    nobsT = nobsT_ref[...]
    we = we_ref[...]
    be = be_ref[...]
    m = jnp.zeros((H, BL), f32)
    nobs_ego = jnp.zeros((F, BL), f32)
    for n in range(N):
        h_n = jax.nn.relu(_f32dot(we, nobsT[n], ((0,), (0,))) + be)
        m = m + aT[n:n + 1, :] * h_n
        nobs_ego = nobs_ego + mask[n:n + 1, :] * nobsT[n]
    ego = jax.nn.relu(_f32dot(we, nobs_ego, ((0,), (0,))) + be)

    p = jnp.concatenate([m, ego], axis=0)
    nbd = jax.nn.relu(_f32dot(p, wms_ref[...], ((0,), (0,))))

    x = jax.nn.relu(
        _f32dot(cent_ref[...], w1c_ref[...], ((1,), (0,)))
        + _f32dot(nbd, w1h_ref[...], ((1,), (0,)))
        + b1_ref[...]
    )
    x = jax.nn.relu(_f32dot(x, w2_ref[...], ((1,), (0,))) + b2_ref[...])
    out_ref[...] = jnp.sum(x * wv_ref[...], axis=1, keepdims=True) + bv_ref[...]


def kernel(cent_obs, node_obs, adj, agent_id, rnn_states, masks,
           W_embed, b_embed, W_msg, W_self, W1, b1, W2, b2, Wv, bv):
    adjT = jnp.transpose(adj, (1, 2, 0))
    nobsT = jnp.transpose(node_obs, (1, 2, 0))
    aidT = agent_id.astype(jnp.int32).reshape(1, B)
    wms = jnp.concatenate([W_msg, W_self], axis=0)

    grid = (B // BL,)
    full = lambda *s: pl.BlockSpec(s, lambda i: (0,) * len(s))
    values = pl.pallas_call(
        _tc_body,
        grid=grid,
        in_specs=[
            pl.BlockSpec((N, N, BL), lambda i: (0, 0, i)),
            pl.BlockSpec((N, F, BL), lambda i: (0, 0, i)),
            pl.BlockSpec((1, BL), lambda i: (0, i)),
            pl.BlockSpec((BL, C), lambda i: (i, 0)),
            full(F, H),
            full(H, 1),
            full(2 * H, H),
            full(C, H),
            full(H, H),
            full(1, H),
            full(H, H),
            full(1, H),
            full(1, H),
            full(1, 1),
        ],
        out_specs=pl.BlockSpec((BL, 1), lambda i: (i, 0)),
        out_shape=jax.ShapeDtypeStruct((B, 1), jnp.float32),
    )(adjT, nobsT, aidT, cent_obs,
      W_embed, b_embed.reshape(H, 1), wms,
      W1[:C], W1[C:], b1.reshape(1, H), W2, b2.reshape(1, H),
      Wv.reshape(1, H), bv.reshape(1, 1))
    return values, rnn_states
